# baseline (speedup 1.0000x reference)
import functools

import jax
import jax.numpy as jnp
from jax import lax
from jax.experimental import pallas as pl
from jax.experimental.pallas import tpu as pltpu
from jax.experimental.pallas import tpu_sc as plsc

_HID = 64
_HEADS = 4
_D = _HEADS * _HID
_N = 50000
_NP = 50176
_RPT = _NP // 16
_E = 850000
_CA = 112
_ITA = 240
_CG = 128
_ITG = 210
_NW = 32
_EP = _NW * _ITA * _CA
_BLK = 512
_NH = 25088


def _ln(x, g, b, eps=1e-5):
    m = x.mean(-1, keepdims=True)
    v = ((x - m) ** 2).mean(-1, keepdims=True)
    return (x - m) / jnp.sqrt(v + eps) * g + b


def _elu(x):
    return jnp.where(x > 0, x, jnp.exp(jnp.minimum(x, 0.0)) - 1.0)


def _proj_body(x_ref, w_ref, b_ref, g_ref, bb_ref, o_ref):
    y = jnp.dot(x_ref[...], w_ref[...], preferred_element_type=jnp.float32)
    y = y + b_ref[...][None, :]
    o_ref[...] = _elu(_ln(y, g_ref[...][None, :], bb_ref[...][None, :]))


def _proj(x_all, w, b, g, bb):
    n, k = x_all.shape
    k_pad = (-k) % 128
    xp = jnp.pad(x_all, ((0, _NP - n), (0, k_pad)))
    wp = jnp.pad(w, ((0, k_pad), (0, 0)))
    return pl.pallas_call(
        _proj_body,
        grid=(_NP // _BLK,),
        in_specs=[
            pl.BlockSpec((_BLK, k + k_pad), lambda i: (i, 0)),
            pl.BlockSpec((k + k_pad, _HID), lambda i: (0, 0)),
            pl.BlockSpec((_HID,), lambda i: (0,)),
            pl.BlockSpec((_HID,), lambda i: (0,)),
            pl.BlockSpec((_HID,), lambda i: (0,)),
        ],
        out_specs=pl.BlockSpec((_BLK, _HID), lambda i: (i, 0)),
        out_shape=jax.ShapeDtypeStruct((_NP, _HID), jnp.float32),
    )(xp, wp, b, g, bb)


def _mm_body(x_ref, w_ref, o_ref):
    o_ref[...] = jnp.dot(x_ref[...], w_ref[...], preferred_element_type=jnp.float32)


def _mm_cols_body(x_ref, w_ref, o_ref):
    o_ref[...] = jnp.dot(x_ref[...], w_ref[0], preferred_element_type=jnp.float32)


def _mm_cols(x, w):
    w_cols = w.reshape(_HID, 8, 32).transpose(1, 0, 2)
    return pl.pallas_call(
        _mm_cols_body,
        grid=(8, _NP // _BLK),
        in_specs=[
            pl.BlockSpec((_BLK, _HID), lambda p, i: (i, 0)),
            pl.BlockSpec((1, _HID, 32), lambda p, i: (p, 0, 0)),
        ],
        out_specs=pl.BlockSpec(
            (_BLK, 32), lambda p, i: (p * (_NP // _BLK) + i, 0)),
        out_shape=jax.ShapeDtypeStruct((8 * _NP, 32), jnp.float32),
    )(x, w_cols)


def _mm(x, w):
    return pl.pallas_call(
        _mm_body,
        grid=(_NP // _BLK,),
        in_specs=[
            pl.BlockSpec((_BLK, _HID), lambda i: (i, 0)),
            pl.BlockSpec((_HID, _D), lambda i: (0, 0)),
        ],
        out_specs=pl.BlockSpec((_BLK, _D), lambda i: (i, 0)),
        out_shape=jax.ShapeDtypeStruct((_NP, _D), jnp.float32),
    )(x, w)


def _epi_body(num_ref, den_ref, x_ref, gb_ref, g_ref, b_ref, o_ref):
    nn = num_ref[0] + num_ref[1]
    d = den_ref[0] + den_ref[1]
    halves = []
    for q in range(2):
        m = None
        for h in range(_HEADS):
            inv = 1.0 / (d[:, 8 * h:8 * h + 1] + 1e-16)
            t = nn[2 * h + q] * inv
            m = t if m is None else m + t
        halves.append(m)
    m = jnp.concatenate(halves, axis=1) * (1.0 / _HEADS) + gb_ref[...][None, :]
    m = _elu(m) + x_ref[...]
    o_ref[...] = _ln(m, g_ref[...][None, :], b_ref[...][None, :])


def _epilogue(num, den, x_res, gat_bias, ln_g, ln_b):
    return pl.pallas_call(
        _epi_body,
        grid=(_NP // _BLK,),
        in_specs=[
            pl.BlockSpec((2, 8, _BLK, 32), lambda i: (0, 0, i, 0)),
            pl.BlockSpec((2, _BLK, 32), lambda i: (0, i, 0)),
            pl.BlockSpec((_BLK, _HID), lambda i: (i, 0)),
            pl.BlockSpec((_HID,), lambda i: (0,)),
            pl.BlockSpec((_HID,), lambda i: (0,)),
            pl.BlockSpec((_HID,), lambda i: (0,)),
        ],
        out_specs=pl.BlockSpec((_BLK, _HID), lambda i: (i, 0)),
        out_shape=jax.ShapeDtypeStruct((_NP, _HID), jnp.float32),
    )(num, den, x_res, gat_bias, ln_g, ln_b)


def _head_body(h_ref, w1_ref, b1_ref, w2_ref, b2_ref, o_ref):
    y = jnp.dot(h_ref[...], w1_ref[...], preferred_element_type=jnp.float32)
    y = _elu(y + b1_ref[...][None, :])
    z = jnp.sum(y * w2_ref[...], axis=1, keepdims=True) + b2_ref[...][None, :]
    o_ref[...] = 1.0 / (1.0 + jnp.exp(-z))


def _head(h, w1, b1, w2r, b2):
    return pl.pallas_call(
        _head_body,
        grid=(_NH // _BLK,),
        in_specs=[
            pl.BlockSpec((_BLK, _HID), lambda i: (i, 0)),
            pl.BlockSpec((_HID, 32), lambda i: (0, 0)),
            pl.BlockSpec((32,), lambda i: (0,)),
            pl.BlockSpec((1, 32), lambda i: (0, 0)),
            pl.BlockSpec((1,), lambda i: (0,)),
        ],
        out_specs=pl.BlockSpec((_BLK, 1), lambda i: (i, 0)),
        out_shape=jax.ShapeDtypeStruct((_NH, 1), jnp.float32),
    )(h, w1, b1, w2r, b2)


_MESH = plsc.VectorSubcoreMesh(core_axis_name="c", subcore_axis_name="s")
_BISECT_AGG_XLA = False


@functools.partial(
    pl.kernel,
    out_type=jax.ShapeDtypeStruct((_EP * 4,), jnp.float32),
    mesh=_MESH,
    compiler_params=pltpu.CompilerParams(needs_layout_passes=False),
    scratch_types=[
        pltpu.VMEM((2, _CA), jnp.int32),
        pltpu.VMEM((2, _CA), jnp.int32),
        pltpu.VMEM((2, _CA, _D), jnp.float32),
        pltpu.VMEM((2, _CA, _D), jnp.float32),
        pltpu.VMEM((_CA * 4,), jnp.float32),
        pltpu.VMEM((_D,), jnp.float32),
        [pltpu.SemaphoreType.DMA] * 2,
        [pltpu.SemaphoreType.DMA] * 2,
        [pltpu.SemaphoreType.DMA] * 2,
        [pltpu.SemaphoreType.DMA] * 2,
    ],
)
def _alpha_kernel(xl_hbm, xr_hbm, src_hbm, dst_hbm, att_hbm, alpha_out,
                  srcv, dstv, xlg, xrg, astage, attv,
                  ssem, dsem, lsem, rsem):
    c = lax.axis_index("c")
    s = lax.axis_index("s")
    wid = s * 2 + c
    pltpu.sync_copy(att_hbm, attv)
    att_regs = [attv[pl.ds(k * 16, 16)] for k in range(16)]
    lane = lax.iota(jnp.int32, 16)

    def idx_start(cj, b):
        base = (wid * _ITA + cj) * _CA
        pltpu.async_copy(src_hbm.at[pl.ds(base, _CA)], srcv.at[b], ssem[b])
        pltpu.async_copy(dst_hbm.at[pl.ds(base, _CA)], dstv.at[b], dsem[b])

    def idx_wait(b):
        pltpu.make_async_copy(src_hbm.at[pl.ds(0, _CA)], srcv.at[b],
                              ssem[b]).wait()
        pltpu.make_async_copy(dst_hbm.at[pl.ds(0, _CA)], dstv.at[b],
                              dsem[b]).wait()

    def gather_start(b):
        pltpu.async_copy(xl_hbm.at[srcv.at[b]], xlg.at[b], lsem[b])
        pltpu.async_copy(xr_hbm.at[dstv.at[b]], xrg.at[b], rsem[b])

    def gather_wait(b):
        pltpu.make_async_copy(xl_hbm.at[srcv.at[b]], xlg.at[b],
                              lsem[b]).wait()
        pltpu.make_async_copy(xr_hbm.at[dstv.at[b]], xrg.at[b],
                              rsem[b]).wait()

    idx_start(0, 0)
    idx_wait(0)
    gather_start(0)
    idx_start(1, 1)

    def pair(j2, carry):
        for half in (0, 1):
            cj = 2 * j2 + half
            nb = 1 - half

            @pl.when(cj + 1 < _ITA)
            def _():
                idx_wait(nb)
                gather_start(nb)

            gather_wait(half)

            @pl.when(cj + 2 < _ITA)
            def _():
                idx_start(cj + 2, half)

            xlgh = xlg.at[half]
            xrgh = xrg.at[half]

            def group(g, carry2):
                vec = jnp.zeros((16,), jnp.float32)
                for ee in range(4):
                    e = g * 4 + ee
                    for h in range(_HEADS):
                        acc = jnp.zeros((16,), jnp.float32)
                        for j in range(4):
                            k = h * 4 + j
                            z = (xlgh[e, pl.ds(k * 16, 16)]
                                 + xrgh[e, pl.ds(k * 16, 16)])
                            z = jnp.where(z > 0, z, 0.2 * z)
                            acc = acc + z * att_regs[k]
                        a = jnp.sum(acc)
                        vec = jnp.where(lane == (ee * 4 + h), a, vec)
                astage[pl.ds(g * 16, 16)] = vec
                return carry2

            lax.fori_loop(0, _CA // 4, group, 0)
            base = (wid * _ITA + cj) * _CA
            pltpu.sync_copy(astage, alpha_out.at[pl.ds(base * 4, _CA * 4)])
        return carry

    lax.fori_loop(0, _ITA // 2, pair, 0)


@functools.partial(
    pl.kernel,
    out_type=(
        jax.ShapeDtypeStruct((16 * _NP, 32), jnp.float32),
        jax.ShapeDtypeStruct((2 * _NP, 32), jnp.float32),
    ),
    mesh=_MESH,
    compiler_params=pltpu.CompilerParams(
        needs_layout_passes=False, use_tc_tiling_on_sc=False),
    scratch_types=[
        pltpu.VMEM((2, _CG), jnp.int32),
        pltpu.VMEM((2, _CG), jnp.int32),
        pltpu.VMEM((2, _CG), jnp.int32),
        pltpu.VMEM((2, _CG * 4), jnp.float32),
        pltpu.VMEM((2, _CG, 32), jnp.float32),
        pltpu.VMEM((2, _CG, 32), jnp.float32),
        pltpu.VMEM((_RPT // 32, 32), jnp.float32),
        pltpu.VMEM((16,), jnp.float32),
        pltpu.VMEM_SHARED((_NP, 32), jnp.float32),
        [pltpu.SemaphoreType.DMA] * 2,
        [pltpu.SemaphoreType.DMA] * 2,
        [pltpu.SemaphoreType.DMA] * 2,
        [pltpu.SemaphoreType.DMA] * 2,
        [pltpu.SemaphoreType.DMA] * 2,
    ],
)
def _agg_kernel(xlcols_hbm, src_hbm, dst_hbm, alpha_hbm, gmax_hbm,
                num_out, den_out,
                srcv, dstv, sidx, wv, xg, sst, zbuf, gv, acc,
                ssem, dsem, asem, gsem, wsem):
    c = lax.axis_index("c")
    s = lax.axis_index("s")
    wid = s * 2 + c
    zrows = _RPT // 32
    pltpu.sync_copy(gmax_hbm, gv)
    gmax_vec = gv[...]
    lane = lax.iota(jnp.int32, 16)
    rep = lane // 8

    def zb(i, carry):
        zbuf[i, pl.ds(0, 16)] = jnp.zeros((16,), jnp.float32)
        zbuf[i, pl.ds(16, 16)] = jnp.zeros((16,), jnp.float32)
        return carry

    lax.fori_loop(0, zrows, zb, 0)

    for p in range(9):
        def idx_start(cj, b, p=p):
            base = (wid * _ITG + cj) * _CG
            pltpu.async_copy(dst_hbm.at[pl.ds(base, _CG)], dstv.at[b],
                             dsem[b])
            pltpu.async_copy(alpha_hbm.at[pl.ds(base * 4, _CG * 4)],
                             wv.at[b], asem[b])
            if p < 8:
                pltpu.async_copy(src_hbm.at[pl.ds(base, _CG)], srcv.at[b],
                                 ssem[b])

        def idx_wait(b, p=p):
            pltpu.make_async_copy(dst_hbm.at[pl.ds(0, _CG)], dstv.at[b],
                                  dsem[b]).wait()
            pltpu.make_async_copy(alpha_hbm.at[pl.ds(0, _CG * 4)],
                                  wv.at[b], asem[b]).wait()
            if p < 8:
                pltpu.make_async_copy(src_hbm.at[pl.ds(0, _CG)],
                                      srcv.at[b], ssem[b]).wait()

        def addoff(b, p=p):
            def go(j, carry):
                srcv[b, pl.ds(j * 16, 16)] = (
                    srcv[b, pl.ds(j * 16, 16)] + (p * _NP))
                return carry
            lax.fori_loop(0, _CG // 16, go, 0)

        def gather_start(b):
            pltpu.async_copy(xlcols_hbm.at[srcv.at[b]], xg.at[b], gsem[b])

        def gather_wait(b):
            pltpu.make_async_copy(xlcols_hbm.at[srcv.at[b]], xg.at[b],
                                  gsem[b]).wait()

        def zr(i, carry):
            pltpu.sync_copy(zbuf, acc.at[pl.ds(s * _RPT + i * zrows, zrows)])
            return carry

        lax.fori_loop(0, 32, zr, 0)
        plsc.subcore_barrier()

        idx_start(0, 0)
        idx_wait(0)
        if p < 8:
            addoff(0)
            gather_start(0)
        idx_start(1, 1)

        def pair(j2, carry, p=p, idx_start=idx_start, idx_wait=idx_wait,
                 addoff=addoff, gather_start=gather_start,
                 gather_wait=gather_wait):
            for half in (0, 1):
                cj = 2 * j2 + half
                nb = 1 - half

                @pl.when(cj + 1 < _ITG)
                def _():
                    idx_wait(nb)
                    if p < 8:
                        addoff(nb)
                        gather_start(nb)

                @pl.when(cj >= 2)
                def _():
                    pltpu.make_async_copy(
                        sst.at[half], acc.at[sidx.at[half]],
                        wsem[half]).wait()

                ssth = sst.at[half]
                if p < 8:
                    gather_wait(half)
                    h = p // 2
                    wvh = wv.at[half]
                    xgh = xg.at[half]

                    def scale(g, carry2):
                        wvec = jnp.exp(wvh[pl.ds(g * 16, 16)] - gmax_vec)
                        for ee in range(4):
                            e = g * 4 + ee
                            wsc = wvec[ee * 4 + h]
                            ssth[e, pl.ds(0, 16)] = xgh[e, pl.ds(0, 16)] * wsc
                            ssth[e, pl.ds(16, 16)] = xgh[e, pl.ds(16, 16)] * wsc
                        return carry2

                    lax.fori_loop(0, _CG // 4, scale, 0)
                else:
                    wvh = wv.at[half]

                    def expv(j, carry2):
                        wvh[pl.ds(j * 16, 16)] = jnp.exp(
                            wvh[pl.ds(j * 16, 16)] - gmax_vec)
                        return carry2

                    lax.fori_loop(0, _CG * 4 // 16, expv, 0)

                    def dstg(e, carry2):
                        ia = e * 4 + rep
                        ssth[e, pl.ds(0, 16)] = plsc.load_gather(wvh, [ia])
                        ssth[e, pl.ds(16, 16)] = plsc.load_gather(
                            wvh, [ia + 2])
                        return carry2

                    lax.fori_loop(0, _CG, dstg, 0)

                def cpidx(j, carry2):
                    sidx[half, pl.ds(j * 16, 16)] = dstv[half, pl.ds(j * 16, 16)]
                    return carry2

                lax.fori_loop(0, _CG // 16, cpidx, 0)
                pltpu.async_copy(sst.at[half], acc.at[sidx.at[half]],
                                 wsem[half], add=True)

                @pl.when(cj + 2 < _ITG)
                def _():
                    idx_start(cj + 2, half)
            return carry

        lax.fori_loop(0, _ITG // 2, pair, 0)
        for b in (0, 1):
            pltpu.make_async_copy(sst.at[b], acc.at[sidx.at[b]],
                                  wsem[b]).wait()
        plsc.subcore_barrier()
        if p < 8:
            row0 = (c * 8 + p) * _NP + s * _RPT
            pltpu.sync_copy(acc.at[pl.ds(s * _RPT, _RPT)],
                            num_out.at[pl.ds(row0, _RPT)])
        else:
            row0 = c * _NP + s * _RPT
            pltpu.sync_copy(acc.at[pl.ds(s * _RPT, _RPT)],
                            den_out.at[pl.ds(row0, _RPT)])
        plsc.subcore_barrier()


def _gat_layer(x, srcp, dstp, wl, wr, att, bias, ln_g, ln_b):
    xl = _mm(x, wl)
    xr = _mm(x, wr)
    xlcols = _mm_cols(x, wl)
    alpha = _alpha_kernel(xl, xr, srcp, dstp, att.reshape(-1))
    gmax = jnp.max(alpha)
    gv = jnp.full((16,), gmax, jnp.float32)
    if _BISECT_AGG_XLA:
        w = jnp.exp(alpha.reshape(_EP, 4) - gmax)
        den_x = jax.ops.segment_sum(w, dstp, num_segments=_NP)
        xlr = xl.reshape(_NP, 4, 64)
        num_x = jax.ops.segment_sum(
            xlr[srcp] * w[:, :, None], dstp, num_segments=_NP)
        num = jnp.zeros((16 * _NP, 32), jnp.float32)
        numr = num_x.reshape(_NP, 8, 32).transpose(1, 0, 2).reshape(8 * _NP, 32)
        num = num.at[: 8 * _NP].set(numr)
        den = jnp.zeros((2 * _NP, 32), jnp.float32)
        den = den.at[:_NP].set(jnp.repeat(den_x, 8, axis=1))
    else:
        num, den = _agg_kernel(xlcols, srcp, dstp, alpha, gv)
    return _epilogue(num.reshape(2, 8, _NP, 32), den.reshape(2, _NP, 32),
                     x, bias, ln_g, ln_b)


def kernel(x_reaction, x_metabolite, edge_rxn2met, edge_met2rxn, proj_w, proj_b, proj_ln_g, proj_ln_b, g1_wl, g1_wr, g1_att, g1_bias, ln1_g, ln1_b, g2_wl, g2_wr, g2_att, g2_bias, ln2_g, ln2_b, head_w1, head_b1, head_w2, head_b2):
    n_rxn = x_reaction.shape[0]
    x_all = jnp.concatenate([x_reaction, x_metabolite], axis=0)
    x = _proj(x_all, proj_w, proj_b, proj_ln_g, proj_ln_b)

    loop = jnp.arange(_N, dtype=jnp.int32)
    pad = jnp.full((_EP - _E,), _N, jnp.int32)
    srcp = jnp.concatenate([
        edge_rxn2met[0].astype(jnp.int32),
        edge_met2rxn[0].astype(jnp.int32) + n_rxn,
        loop, pad,
    ])
    dstp = jnp.concatenate([
        edge_rxn2met[1].astype(jnp.int32) + n_rxn,
        edge_met2rxn[1].astype(jnp.int32),
        loop, pad,
    ])

    h = _gat_layer(x, srcp, dstp, g1_wl, g1_wr, g1_att, g1_bias, ln1_g, ln1_b)
    h = _gat_layer(h, srcp, dstp, g2_wl, g2_wr, g2_att, g2_bias, ln2_g, ln2_b)

    y = _head(h[:_NH], head_w1, head_b1, head_w2.reshape(1, 32), head_b2)
    return y[:n_rxn, 0]


# speedup vs baseline: 1.0063x; 1.0063x over previous
import functools

import jax
import jax.numpy as jnp
from jax import lax
from jax.experimental import pallas as pl
from jax.experimental.pallas import tpu as pltpu
from jax.experimental.pallas import tpu_sc as plsc

_HID = 64
_HEADS = 4
_D = _HEADS * _HID
_N = 50000
_NP = 50176
_RPT = _NP // 16
_E = 850000
_CA = 112
_ITA = 240
_CG = 128
_ITG = 210
_NW = 32
_EP = _NW * _ITA * _CA
_BLK = 512
_NH = 25088


def _ln(x, g, b, eps=1e-5):
    m = x.mean(-1, keepdims=True)
    v = ((x - m) ** 2).mean(-1, keepdims=True)
    return (x - m) / jnp.sqrt(v + eps) * g + b


def _elu(x):
    return jnp.where(x > 0, x, jnp.exp(jnp.minimum(x, 0.0)) - 1.0)


def _proj_body(x_ref, w_ref, b_ref, g_ref, bb_ref, o_ref):
    y = jnp.dot(x_ref[...], w_ref[...], preferred_element_type=jnp.float32)
    y = y + b_ref[...][None, :]
    o_ref[...] = _elu(_ln(y, g_ref[...][None, :], bb_ref[...][None, :]))


def _proj(x_all, w, b, g, bb):
    n, k = x_all.shape
    k_pad = (-k) % 128
    xp = jnp.pad(x_all, ((0, _NP - n), (0, k_pad)))
    wp = jnp.pad(w, ((0, k_pad), (0, 0)))
    return pl.pallas_call(
        _proj_body,
        grid=(_NP // _BLK,),
        in_specs=[
            pl.BlockSpec((_BLK, k + k_pad), lambda i: (i, 0)),
            pl.BlockSpec((k + k_pad, _HID), lambda i: (0, 0)),
            pl.BlockSpec((_HID,), lambda i: (0,)),
            pl.BlockSpec((_HID,), lambda i: (0,)),
            pl.BlockSpec((_HID,), lambda i: (0,)),
        ],
        out_specs=pl.BlockSpec((_BLK, _HID), lambda i: (i, 0)),
        out_shape=jax.ShapeDtypeStruct((_NP, _HID), jnp.float32),
    )(xp, wp, b, g, bb)


def _mm_body(x_ref, w_ref, o_ref):
    o_ref[...] = jnp.dot(x_ref[...], w_ref[...], preferred_element_type=jnp.float32)


def _mm(x, w):
    return pl.pallas_call(
        _mm_body,
        grid=(_NP // _BLK,),
        in_specs=[
            pl.BlockSpec((_BLK, _HID), lambda i: (i, 0)),
            pl.BlockSpec((_HID, _D), lambda i: (0, 0)),
        ],
        out_specs=pl.BlockSpec((_BLK, _D), lambda i: (i, 0)),
        out_shape=jax.ShapeDtypeStruct((_NP, _D), jnp.float32),
    )(x, w)


def _epi_body(num_ref, den_ref, x_ref, gb_ref, g_ref, b_ref, o_ref):
    nn = num_ref[0] + num_ref[1]
    d = den_ref[0] + den_ref[1]
    halves = []
    for q in range(2):
        m = None
        for h in range(_HEADS):
            inv = 1.0 / (d[:, 8 * h:8 * h + 1] + 1e-16)
            t = nn[2 * h + q] * inv
            m = t if m is None else m + t
        halves.append(m)
    m = jnp.concatenate(halves, axis=1) * (1.0 / _HEADS) + gb_ref[...][None, :]
    m = _elu(m) + x_ref[...]
    o_ref[...] = _ln(m, g_ref[...][None, :], b_ref[...][None, :])


def _epilogue(num, den, x_res, gat_bias, ln_g, ln_b):
    return pl.pallas_call(
        _epi_body,
        grid=(_NP // _BLK,),
        in_specs=[
            pl.BlockSpec((2, 8, _BLK, 32), lambda i: (0, 0, i, 0)),
            pl.BlockSpec((2, _BLK, 32), lambda i: (0, i, 0)),
            pl.BlockSpec((_BLK, _HID), lambda i: (i, 0)),
            pl.BlockSpec((_HID,), lambda i: (0,)),
            pl.BlockSpec((_HID,), lambda i: (0,)),
            pl.BlockSpec((_HID,), lambda i: (0,)),
        ],
        out_specs=pl.BlockSpec((_BLK, _HID), lambda i: (i, 0)),
        out_shape=jax.ShapeDtypeStruct((_NP, _HID), jnp.float32),
    )(num, den, x_res, gat_bias, ln_g, ln_b)


def _head_body(h_ref, w1_ref, b1_ref, w2_ref, b2_ref, o_ref):
    y = jnp.dot(h_ref[...], w1_ref[...], preferred_element_type=jnp.float32)
    y = _elu(y + b1_ref[...][None, :])
    z = jnp.sum(y * w2_ref[...], axis=1, keepdims=True) + b2_ref[...][None, :]
    o_ref[...] = 1.0 / (1.0 + jnp.exp(-z))


def _head(h, w1, b1, w2r, b2):
    return pl.pallas_call(
        _head_body,
        grid=(_NH // _BLK,),
        in_specs=[
            pl.BlockSpec((_BLK, _HID), lambda i: (i, 0)),
            pl.BlockSpec((_HID, 32), lambda i: (0, 0)),
            pl.BlockSpec((32,), lambda i: (0,)),
            pl.BlockSpec((1, 32), lambda i: (0, 0)),
            pl.BlockSpec((1,), lambda i: (0,)),
        ],
        out_specs=pl.BlockSpec((_BLK, 1), lambda i: (i, 0)),
        out_shape=jax.ShapeDtypeStruct((_NH, 1), jnp.float32),
    )(h, w1, b1, w2r, b2)


_MESH = plsc.VectorSubcoreMesh(core_axis_name="c", subcore_axis_name="s")


@functools.partial(
    pl.kernel,
    out_type=jax.ShapeDtypeStruct((_EP * 4,), jnp.float32),
    mesh=_MESH,
    compiler_params=pltpu.CompilerParams(needs_layout_passes=False),
    scratch_types=[
        pltpu.VMEM((2, _CA), jnp.int32),
        pltpu.VMEM((2, _CA), jnp.int32),
        pltpu.VMEM((2, _CA, _D), jnp.float32),
        pltpu.VMEM((2, _CA, _D), jnp.float32),
        pltpu.VMEM((_CA * 4,), jnp.float32),
        pltpu.VMEM((_D,), jnp.float32),
        [pltpu.SemaphoreType.DMA] * 2,
        [pltpu.SemaphoreType.DMA] * 2,
        [pltpu.SemaphoreType.DMA] * 2,
        [pltpu.SemaphoreType.DMA] * 2,
    ],
)
def _alpha_kernel(xl_hbm, xr_hbm, src_hbm, dst_hbm, att_hbm, alpha_out,
                  srcv, dstv, xlg, xrg, astage, attv,
                  ssem, dsem, lsem, rsem):
    c = lax.axis_index("c")
    s = lax.axis_index("s")
    wid = s * 2 + c
    pltpu.sync_copy(att_hbm, attv)
    att_regs = [attv[pl.ds(k * 16, 16)] for k in range(16)]
    lane = lax.iota(jnp.int32, 16)

    def idx_start(cj, b):
        base = (wid * _ITA + cj) * _CA
        pltpu.async_copy(src_hbm.at[pl.ds(base, _CA)], srcv.at[b], ssem[b])
        pltpu.async_copy(dst_hbm.at[pl.ds(base, _CA)], dstv.at[b], dsem[b])

    def idx_wait(b):
        pltpu.make_async_copy(src_hbm.at[pl.ds(0, _CA)], srcv.at[b],
                              ssem[b]).wait()
        pltpu.make_async_copy(dst_hbm.at[pl.ds(0, _CA)], dstv.at[b],
                              dsem[b]).wait()

    def gather_start(b):
        pltpu.async_copy(xl_hbm.at[srcv.at[b]], xlg.at[b], lsem[b])
        pltpu.async_copy(xr_hbm.at[dstv.at[b]], xrg.at[b], rsem[b])

    def gather_wait(b):
        pltpu.make_async_copy(xl_hbm.at[srcv.at[b]], xlg.at[b],
                              lsem[b]).wait()
        pltpu.make_async_copy(xr_hbm.at[dstv.at[b]], xrg.at[b],
                              rsem[b]).wait()

    idx_start(0, 0)
    idx_wait(0)
    gather_start(0)
    idx_start(1, 1)

    def pair(j2, carry):
        for half in (0, 1):
            cj = 2 * j2 + half
            nb = 1 - half

            @pl.when(cj + 1 < _ITA)
            def _():
                idx_wait(nb)
                gather_start(nb)

            gather_wait(half)

            @pl.when(cj + 2 < _ITA)
            def _():
                idx_start(cj + 2, half)

            xlgh = xlg.at[half]
            xrgh = xrg.at[half]

            def group(g, carry2):
                vec = jnp.zeros((16,), jnp.float32)
                for ee in range(4):
                    e = g * 4 + ee
                    for h in range(_HEADS):
                        acc = jnp.zeros((16,), jnp.float32)
                        for j in range(4):
                            k = h * 4 + j
                            z = (xlgh[e, pl.ds(k * 16, 16)]
                                 + xrgh[e, pl.ds(k * 16, 16)])
                            z = jnp.where(z > 0, z, 0.2 * z)
                            acc = acc + z * att_regs[k]
                        a = jnp.sum(acc)
                        vec = jnp.where(lane == (ee * 4 + h), a, vec)
                astage[pl.ds(g * 16, 16)] = vec
                return carry2

            lax.fori_loop(0, _CA // 4, group, 0)
            base = (wid * _ITA + cj) * _CA
            pltpu.sync_copy(astage, alpha_out.at[pl.ds(base * 4, _CA * 4)])
        return carry

    lax.fori_loop(0, _ITA // 2, pair, 0)


@functools.partial(
    pl.kernel,
    out_type=(
        jax.ShapeDtypeStruct((16 * _NP, 32), jnp.float32),
        jax.ShapeDtypeStruct((2 * _NP, 32), jnp.float32),
    ),
    mesh=_MESH,
    compiler_params=pltpu.CompilerParams(
        needs_layout_passes=False, use_tc_tiling_on_sc=False),
    scratch_types=[
        pltpu.VMEM((2, _CG), jnp.int32),
        pltpu.VMEM((2, _CG), jnp.int32),
        pltpu.VMEM((2, _CG), jnp.int32),
        pltpu.VMEM((2, _CG * 4), jnp.float32),
        pltpu.VMEM((2, _CG, 32), jnp.float32),
        pltpu.VMEM((2, _CG, 32), jnp.float32),
        pltpu.VMEM((_RPT // 32, 32), jnp.float32),
        pltpu.VMEM((16,), jnp.float32),
        pltpu.VMEM_SHARED((_NP, 32), jnp.float32),
        [pltpu.SemaphoreType.DMA] * 2,
        [pltpu.SemaphoreType.DMA] * 2,
        [pltpu.SemaphoreType.DMA] * 2,
        [pltpu.SemaphoreType.DMA] * 2,
        [pltpu.SemaphoreType.DMA] * 2,
    ],
)
def _agg_kernel(xlcols_hbm, src_hbm, dst_hbm, alpha_hbm, gmax_hbm,
                num_out, den_out,
                srcv, dstv, sidx, wv, xg, sst, zbuf, gv, acc,
                ssem, dsem, asem, gsem, wsem):
    c = lax.axis_index("c")
    s = lax.axis_index("s")
    wid = s * 2 + c
    zrows = _RPT // 32
    pltpu.sync_copy(gmax_hbm, gv)
    gmax_vec = gv[...]
    lane = lax.iota(jnp.int32, 16)
    rep = lane // 8

    def zb(i, carry):
        zbuf[i, pl.ds(0, 16)] = jnp.zeros((16,), jnp.float32)
        zbuf[i, pl.ds(16, 16)] = jnp.zeros((16,), jnp.float32)
        return carry

    lax.fori_loop(0, zrows, zb, 0)

    for p in range(9):
        def idx_start(cj, b, p=p):
            base = (wid * _ITG + cj) * _CG
            pltpu.async_copy(dst_hbm.at[pl.ds(base, _CG)], dstv.at[b],
                             dsem[b])
            pltpu.async_copy(alpha_hbm.at[pl.ds(base * 4, _CG * 4)],
                             wv.at[b], asem[b])
            if p < 8:
                pltpu.async_copy(src_hbm.at[pl.ds(base, _CG)], srcv.at[b],
                                 ssem[b])

        def idx_wait(b, p=p):
            pltpu.make_async_copy(dst_hbm.at[pl.ds(0, _CG)], dstv.at[b],
                                  dsem[b]).wait()
            pltpu.make_async_copy(alpha_hbm.at[pl.ds(0, _CG * 4)],
                                  wv.at[b], asem[b]).wait()
            if p < 8:
                pltpu.make_async_copy(src_hbm.at[pl.ds(0, _CG)],
                                      srcv.at[b], ssem[b]).wait()

        def addoff(b, p=p):
            def go(j, carry):
                srcv[b, pl.ds(j * 16, 16)] = (
                    srcv[b, pl.ds(j * 16, 16)] + (p * _NP))
                return carry
            lax.fori_loop(0, _CG // 16, go, 0)

        def gather_start(b):
            pltpu.async_copy(xlcols_hbm.at[srcv.at[b]], xg.at[b], gsem[b])

        def gather_wait(b):
            pltpu.make_async_copy(xlcols_hbm.at[srcv.at[b]], xg.at[b],
                                  gsem[b]).wait()

        def zr(i, carry):
            pltpu.sync_copy(zbuf, acc.at[pl.ds(s * _RPT + i * zrows, zrows)])
            return carry

        lax.fori_loop(0, 32, zr, 0)
        plsc.subcore_barrier()

        idx_start(0, 0)
        idx_wait(0)
        if p < 8:
            addoff(0)
            gather_start(0)
        idx_start(1, 1)

        def pair(j2, carry, p=p, idx_start=idx_start, idx_wait=idx_wait,
                 addoff=addoff, gather_start=gather_start,
                 gather_wait=gather_wait):
            for half in (0, 1):
                cj = 2 * j2 + half
                nb = 1 - half

                @pl.when(cj + 1 < _ITG)
                def _():
                    idx_wait(nb)
                    if p < 8:
                        addoff(nb)
                        gather_start(nb)

                @pl.when(cj >= 2)
                def _():
                    pltpu.make_async_copy(
                        sst.at[half], acc.at[sidx.at[half]],
                        wsem[half]).wait()

                ssth = sst.at[half]
                if p < 8:
                    gather_wait(half)
                    h = p // 2
                    wvh = wv.at[half]
                    xgh = xg.at[half]

                    def scale(g, carry2):
                        wvec = jnp.exp(wvh[pl.ds(g * 16, 16)] - gmax_vec)
                        for ee in range(4):
                            e = g * 4 + ee
                            wsc = wvec[ee * 4 + h]
                            ssth[e, pl.ds(0, 16)] = xgh[e, pl.ds(0, 16)] * wsc
                            ssth[e, pl.ds(16, 16)] = xgh[e, pl.ds(16, 16)] * wsc
                        return carry2

                    lax.fori_loop(0, _CG // 4, scale, 0)
                else:
                    wvh = wv.at[half]

                    def expv(j, carry2):
                        wvh[pl.ds(j * 16, 16)] = jnp.exp(
                            wvh[pl.ds(j * 16, 16)] - gmax_vec)
                        return carry2

                    lax.fori_loop(0, _CG * 4 // 16, expv, 0)

                    def dstg(e, carry2):
                        ia = e * 4 + rep
                        ssth[e, pl.ds(0, 16)] = plsc.load_gather(wvh, [ia])
                        ssth[e, pl.ds(16, 16)] = plsc.load_gather(
                            wvh, [ia + 2])
                        return carry2

                    lax.fori_loop(0, _CG, dstg, 0)

                def cpidx(j, carry2):
                    sidx[half, pl.ds(j * 16, 16)] = dstv[half, pl.ds(j * 16, 16)]
                    return carry2

                lax.fori_loop(0, _CG // 16, cpidx, 0)
                pltpu.async_copy(sst.at[half], acc.at[sidx.at[half]],
                                 wsem[half], add=True)

                @pl.when(cj + 2 < _ITG)
                def _():
                    idx_start(cj + 2, half)
            return carry

        lax.fori_loop(0, _ITG // 2, pair, 0)
        for b in (0, 1):
            pltpu.make_async_copy(sst.at[b], acc.at[sidx.at[b]],
                                  wsem[b]).wait()
        plsc.subcore_barrier()
        if p < 8:
            row0 = (c * 8 + p) * _NP + s * _RPT
            pltpu.sync_copy(acc.at[pl.ds(s * _RPT, _RPT)],
                            num_out.at[pl.ds(row0, _RPT)])
        else:
            row0 = c * _NP + s * _RPT
            pltpu.sync_copy(acc.at[pl.ds(s * _RPT, _RPT)],
                            den_out.at[pl.ds(row0, _RPT)])
        plsc.subcore_barrier()


def _gat_layer(x, srcp, dstp, wl, wr, att, bias, ln_g, ln_b):
    xl = _mm(x, wl)
    xr = _mm(x, wr)
    xlcols = xl.reshape(_NP, 8, 32).transpose(1, 0, 2).reshape(8 * _NP, 32)
    alpha = _alpha_kernel(xl, xr, srcp, dstp, att.reshape(-1))
    gmax = jnp.max(alpha)
    gv = jnp.full((16,), gmax, jnp.float32)
    num, den = _agg_kernel(xlcols, srcp, dstp, alpha, gv)
    return _epilogue(num.reshape(2, 8, _NP, 32), den.reshape(2, _NP, 32),
                     x, bias, ln_g, ln_b)


def kernel(x_reaction, x_metabolite, edge_rxn2met, edge_met2rxn, proj_w, proj_b, proj_ln_g, proj_ln_b, g1_wl, g1_wr, g1_att, g1_bias, ln1_g, ln1_b, g2_wl, g2_wr, g2_att, g2_bias, ln2_g, ln2_b, head_w1, head_b1, head_w2, head_b2):
    n_rxn = x_reaction.shape[0]
    x_all = jnp.concatenate([x_reaction, x_metabolite], axis=0)
    x = _proj(x_all, proj_w, proj_b, proj_ln_g, proj_ln_b)

    loop = jnp.arange(_N, dtype=jnp.int32)
    pad = jnp.full((_EP - _E,), _N, jnp.int32)
    srcp = jnp.concatenate([
        edge_rxn2met[0].astype(jnp.int32),
        edge_met2rxn[0].astype(jnp.int32) + n_rxn,
        loop, pad,
    ])
    dstp = jnp.concatenate([
        edge_rxn2met[1].astype(jnp.int32) + n_rxn,
        edge_met2rxn[1].astype(jnp.int32),
        loop, pad,
    ])

    h = _gat_layer(x, srcp, dstp, g1_wl, g1_wr, g1_att, g1_bias, ln1_g, ln1_b)
    h = _gat_layer(h, srcp, dstp, g2_wl, g2_wr, g2_att, g2_bias, ln2_g, ln2_b)

    y = _head(h[:_NH], head_w1, head_b1, head_w2.reshape(1, 32), head_b2)
    return y[:n_rxn, 0]
